# Initial kernel scaffold; baseline (speedup 1.0000x reference)
import jax
import jax.numpy as jnp
from jax.experimental import pallas as pl


def kernel(x, W_enc, b_enc, W_dec):
    raise NotImplementedError("write your pallas kernel here")


# speedup vs baseline: 2.0243x; 2.0243x over previous
import functools

import jax
import jax.numpy as jnp
from jax.experimental import pallas as pl
from jax.experimental.pallas import tpu as pltpu

TOPK = 64


def _encode_body(x_ref, w_ref, b_ref, code_out, idx_out, val_out, abs_ref,
                 *, k, n_tiles, n_tile, r, d):
    j = pl.program_id(1)
    acc = jax.lax.dot_general(
        x_ref[...], w_ref[...], (((1,), (1,)), ((), ())),
        preferred_element_type=jnp.float32)
    acc = acc + b_ref[...]
    code_out[:, pl.ds(j * n_tile, n_tile)] = acc
    abs_ref[:, pl.ds(j * n_tile, n_tile)] = jnp.abs(acc)

    @pl.when(j == n_tiles - 1)
    def _():
        iota = jax.lax.broadcasted_iota(jnp.int32, (r, d), 1)
        iota_k = jax.lax.broadcasted_iota(jnp.int32, (r, k), 1)

        def body(t, carry):
            vals, idxs = carry
            a = abs_ref[...]
            m = jnp.max(a, axis=1, keepdims=True)
            sel = jnp.min(jnp.where(a == m, iota, d), axis=1, keepdims=True)
            is_sel = iota == sel
            v = jnp.sum(jnp.where(is_sel, code_out[...], 0.0), axis=1,
                        keepdims=True)
            abs_ref[...] = jnp.where(is_sel, -1.0, a)
            vals = jnp.where(iota_k == t, v, vals)
            idxs = jnp.where(iota_k == t, sel, idxs)
            return vals, idxs

        vals, idxs = jax.lax.fori_loop(
            0, k, body,
            (jnp.zeros((r, k), jnp.float32), jnp.zeros((r, k), jnp.int32)))
        val_out[...] = vals
        idx_out[...] = idxs
        code_out[...] = jnp.where(abs_ref[...] == -1.0, code_out[...], 0.0)


def _encode(x, W_enc, b_enc, k):
    b, i_dim = x.shape
    d = W_enc.shape[0]
    r = min(128, b)
    n_tile = min(512, d)
    n_tiles = d // n_tile
    grid = (b // r, n_tiles)
    body = functools.partial(_encode_body, k=k, n_tiles=n_tiles,
                             n_tile=n_tile, r=r, d=d)
    return pl.pallas_call(
        body,
        grid=grid,
        in_specs=[
            pl.BlockSpec((r, i_dim), lambda i, j: (i, 0)),
            pl.BlockSpec((n_tile, i_dim), lambda i, j: (j, 0)),
            pl.BlockSpec((1, n_tile), lambda i, j: (0, j)),
        ],
        out_specs=[
            pl.BlockSpec((r, d), lambda i, j: (i, 0)),
            pl.BlockSpec((r, k), lambda i, j: (i, 0)),
            pl.BlockSpec((r, k), lambda i, j: (i, 0)),
        ],
        out_shape=[
            jax.ShapeDtypeStruct((b, d), jnp.float32),
            jax.ShapeDtypeStruct((b, k), jnp.int32),
            jax.ShapeDtypeStruct((b, k), jnp.float32),
        ],
        scratch_shapes=[pltpu.VMEM((r, d), jnp.float32)],
        compiler_params=pltpu.CompilerParams(
            dimension_semantics=("parallel", "arbitrary")),
    )(x, W_enc, b_enc.reshape(1, d))


def _decode_body(sc_ref, w_ref, out_ref):
    kk = pl.program_id(1)

    @pl.when(kk == 0)
    def _():
        out_ref[...] = jnp.zeros_like(out_ref)

    out_ref[...] = out_ref[...] + jax.lax.dot_general(
        sc_ref[...], w_ref[...], (((1,), (1,)), ((), ())),
        preferred_element_type=jnp.float32)


def _decode(sparse_code, W_dec):
    b, d = sparse_code.shape
    i_dim = W_dec.shape[0]
    r = min(1024, b)
    k_tile = min(512, d)
    grid = (b // r, d // k_tile)
    return pl.pallas_call(
        _decode_body,
        grid=grid,
        in_specs=[
            pl.BlockSpec((r, k_tile), lambda i, kk: (i, kk)),
            pl.BlockSpec((i_dim, k_tile), lambda i, kk: (0, kk)),
        ],
        out_specs=pl.BlockSpec((r, i_dim), lambda i, kk: (i, 0)),
        out_shape=jax.ShapeDtypeStruct((b, i_dim), jnp.float32),
        compiler_params=pltpu.CompilerParams(
            dimension_semantics=("parallel", "arbitrary")),
    )(sparse_code, W_dec)


def kernel(x, W_enc, b_enc, W_dec):
    sparse_code, active_indices, _vals = _encode(x, W_enc, b_enc, TOPK)
    reconstructed_x = _decode(sparse_code, W_dec)
    return reconstructed_x, sparse_code, active_indices


# speedup vs baseline: 2.1378x; 1.0561x over previous
import functools

import jax
import jax.numpy as jnp
from jax.experimental import pallas as pl
from jax.experimental.pallas import tpu as pltpu

TOPK = 64


_CHUNK = 128
_QDEPTH = 8


def _encode_body(x_ref, w_ref, b_ref, code_out, idx_out, val_out,
                 qav_ref, qsv_ref, qix_ref,
                 *, k, n_tiles, n_tile, r, d):
    j = pl.program_id(1)
    acc = jax.lax.dot_general(
        x_ref[...], w_ref[...], (((1,), (1,)), ((), ())),
        preferred_element_type=jnp.float32)
    acc = acc + b_ref[...]
    code_out[:, pl.ds(j * n_tile, n_tile)] = acc

    @pl.when(j == n_tiles - 1)
    def _():
        w = _CHUNK
        nc = d // w
        q_depth = _QDEPTH
        c3 = code_out[...].reshape(r, nc, w)
        iota_w = jax.lax.broadcasted_iota(jnp.int32, (r, nc, w), 2)

        a3 = jnp.abs(c3)
        for q in range(q_depth):
            m = jnp.max(a3, axis=2)
            sel = jnp.min(jnp.where(a3 == m[:, :, None], iota_w, w), axis=2)
            is_sel = iota_w == sel[:, :, None]
            qav_ref[:, q, :] = m
            qsv_ref[:, q, :] = jnp.sum(jnp.where(is_sel, c3, 0.0), axis=2)
            qix_ref[:, q, :] = sel
            a3 = jnp.where(is_sel, -1.0, a3)

        iota_q = jax.lax.broadcasted_iota(jnp.int32, (r, q_depth, nc), 1)
        iota_nc = jax.lax.broadcasted_iota(jnp.int32, (r, nc), 1)
        iota_k = jax.lax.broadcasted_iota(jnp.int32, (r, k), 1)
        qav = qav_ref[...]
        qsv = qsv_ref[...]
        qix = qix_ref[...]

        def body(t, carry):
            head, ptr, vals, idxs, _ = carry
            m = jnp.max(head, axis=1, keepdims=True)
            c = jnp.min(jnp.where(head == m, iota_nc, nc), axis=1,
                        keepdims=True)
            onehot = iota_nc == c
            mask3 = onehot[:, None, :] & (iota_q == ptr[:, None, :])
            v = jnp.sum(jnp.sum(jnp.where(mask3, qsv, 0.0), axis=1), axis=1,
                        keepdims=True)
            wi = jnp.sum(jnp.sum(jnp.where(mask3, qix, 0), axis=1), axis=1,
                         keepdims=True)
            gidx = c * w + wi
            ptr = ptr + onehot.astype(jnp.int32)
            head_new = jnp.max(jnp.where(iota_q == ptr[:, None, :], qav, -1.0),
                               axis=1)
            head = jnp.where(onehot, head_new, head)
            vals = jnp.where(iota_k == t, v, vals)
            idxs = jnp.where(iota_k == t, gidx, idxs)
            return head, ptr, vals, idxs, m

        init = (qav_ref[:, 0, :], jnp.zeros((r, nc), jnp.int32),
                jnp.zeros((r, k), jnp.float32), jnp.zeros((r, k), jnp.int32),
                jnp.zeros((r, 1), jnp.float32))
        _, _, vals, idxs, th = jax.lax.fori_loop(0, k, body, init)
        val_out[...] = vals
        idx_out[...] = idxs
        code = code_out[...]
        code_out[...] = jnp.where(jnp.abs(code) >= th, code, 0.0)


def _encode(x, W_enc, b_enc, k):
    b, i_dim = x.shape
    d = W_enc.shape[0]
    r = min(128, b)
    n_tile = min(256, d)
    n_tiles = d // n_tile
    grid = (b // r, n_tiles)
    body = functools.partial(_encode_body, k=k, n_tiles=n_tiles,
                             n_tile=n_tile, r=r, d=d)
    return pl.pallas_call(
        body,
        grid=grid,
        in_specs=[
            pl.BlockSpec((r, i_dim), lambda i, j: (i, 0)),
            pl.BlockSpec((n_tile, i_dim), lambda i, j: (j, 0)),
            pl.BlockSpec((1, n_tile), lambda i, j: (0, j)),
        ],
        out_specs=[
            pl.BlockSpec((r, d), lambda i, j: (i, 0)),
            pl.BlockSpec((r, k), lambda i, j: (i, 0)),
            pl.BlockSpec((r, k), lambda i, j: (i, 0)),
        ],
        out_shape=[
            jax.ShapeDtypeStruct((b, d), jnp.float32),
            jax.ShapeDtypeStruct((b, k), jnp.int32),
            jax.ShapeDtypeStruct((b, k), jnp.float32),
        ],
        scratch_shapes=[
            pltpu.VMEM((r, _QDEPTH, d // _CHUNK), jnp.float32),
            pltpu.VMEM((r, _QDEPTH, d // _CHUNK), jnp.float32),
            pltpu.VMEM((r, _QDEPTH, d // _CHUNK), jnp.int32),
        ],
        compiler_params=pltpu.CompilerParams(
            dimension_semantics=("parallel", "arbitrary")),
    )(x, W_enc, b_enc.reshape(1, d))


def _decode_body(sc_ref, w_ref, out_ref):
    kk = pl.program_id(1)

    @pl.when(kk == 0)
    def _():
        out_ref[...] = jnp.zeros_like(out_ref)

    out_ref[...] = out_ref[...] + jax.lax.dot_general(
        sc_ref[...], w_ref[...], (((1,), (1,)), ((), ())),
        preferred_element_type=jnp.float32)


def _decode(sparse_code, W_dec):
    b, d = sparse_code.shape
    i_dim = W_dec.shape[0]
    r = min(1024, b)
    k_tile = min(512, d)
    grid = (b // r, d // k_tile)
    return pl.pallas_call(
        _decode_body,
        grid=grid,
        in_specs=[
            pl.BlockSpec((r, k_tile), lambda i, kk: (i, kk)),
            pl.BlockSpec((i_dim, k_tile), lambda i, kk: (0, kk)),
        ],
        out_specs=pl.BlockSpec((r, i_dim), lambda i, kk: (i, 0)),
        out_shape=jax.ShapeDtypeStruct((b, i_dim), jnp.float32),
        compiler_params=pltpu.CompilerParams(
            dimension_semantics=("parallel", "arbitrary")),
    )(sparse_code, W_dec)


def kernel(x, W_enc, b_enc, W_dec):
    sparse_code, active_indices, _vals = _encode(x, W_enc, b_enc, TOPK)
    reconstructed_x = _decode(sparse_code, W_dec)
    return reconstructed_x, sparse_code, active_indices


# speedup vs baseline: 2.9857x; 1.3966x over previous
import functools

import jax
import jax.numpy as jnp
from jax.experimental import pallas as pl
from jax.experimental.pallas import tpu as pltpu

TOPK = 64
_CHUNK = 128
_QDEPTH = 8


def _matmul_body(x_ref, w_ref, b_ref, out_ref):
    out_ref[...] = jax.lax.dot_general(
        x_ref[...], w_ref[...], (((1,), (1,)), ((), ())),
        preferred_element_type=jnp.float32) + b_ref[...]


def _encode_matmul(x, W_enc, b_enc):
    b, i_dim = x.shape
    d = W_enc.shape[0]
    r = min(1024, b)
    n_tile = min(512, d)
    grid = (b // r, d // n_tile)
    return pl.pallas_call(
        _matmul_body,
        grid=grid,
        in_specs=[
            pl.BlockSpec((r, i_dim), lambda i, j: (i, 0)),
            pl.BlockSpec((n_tile, i_dim), lambda i, j: (j, 0)),
            pl.BlockSpec((1, n_tile), lambda i, j: (0, j)),
        ],
        out_specs=pl.BlockSpec((r, n_tile), lambda i, j: (i, j)),
        out_shape=jax.ShapeDtypeStruct((b, d), jnp.float32),
        compiler_params=pltpu.CompilerParams(
            dimension_semantics=("parallel", "arbitrary")),
    )(x, W_enc, b_enc.reshape(1, d))


def _topk_body(code_ref, sc_out, idx_out, val_out, qav_ref, qsv_ref, qix_ref,
               *, k, r, d):
    w = _CHUNK
    nc = d // w
    q_depth = _QDEPTH
    code = code_ref[...]
    c3 = code.reshape(r, nc, w)
    iota_w = jax.lax.broadcasted_iota(jnp.int32, (r, nc, w), 2)

    a3 = jnp.abs(c3)
    for q in range(q_depth):
        m = jnp.max(a3, axis=2)
        sel = jnp.min(jnp.where(a3 == m[:, :, None], iota_w, w), axis=2)
        is_sel = iota_w == sel[:, :, None]
        qav_ref[:, q, :] = m
        qsv_ref[:, q, :] = jnp.sum(jnp.where(is_sel, c3, 0.0), axis=2)
        qix_ref[:, q, :] = sel
        a3 = jnp.where(is_sel, -1.0, a3)

    iota_q = jax.lax.broadcasted_iota(jnp.int32, (r, q_depth, nc), 1)
    iota_nc = jax.lax.broadcasted_iota(jnp.int32, (r, nc), 1)
    iota_k = jax.lax.broadcasted_iota(jnp.int32, (r, k), 1)
    qav = qav_ref[...]
    qsv = qsv_ref[...]
    qix = qix_ref[...]

    def body(t, carry):
        head, ptr, vals, idxs, _ = carry
        m = jnp.max(head, axis=1, keepdims=True)
        c = jnp.min(jnp.where(head == m, iota_nc, nc), axis=1, keepdims=True)
        onehot = iota_nc == c
        mask3 = onehot[:, None, :] & (iota_q == ptr[:, None, :])
        v = jnp.sum(jnp.sum(jnp.where(mask3, qsv, 0.0), axis=1), axis=1,
                    keepdims=True)
        wi = jnp.sum(jnp.sum(jnp.where(mask3, qix, 0), axis=1), axis=1,
                     keepdims=True)
        gidx = c * w + wi
        ptr = ptr + onehot.astype(jnp.int32)
        head_new = jnp.max(jnp.where(iota_q == ptr[:, None, :], qav, -1.0),
                           axis=1)
        head = jnp.where(onehot, head_new, head)
        vals = jnp.where(iota_k == t, v, vals)
        idxs = jnp.where(iota_k == t, gidx, idxs)
        return head, ptr, vals, idxs, m

    init = (qav_ref[:, 0, :], jnp.zeros((r, nc), jnp.int32),
            jnp.zeros((r, k), jnp.float32), jnp.zeros((r, k), jnp.int32),
            jnp.zeros((r, 1), jnp.float32))
    _, _, vals, idxs, th = jax.lax.fori_loop(0, k, body, init)
    val_out[...] = vals
    idx_out[...] = idxs
    sc_out[...] = jnp.where(jnp.abs(code) >= th, code, 0.0)


def _topk(dense_code, k):
    b, d = dense_code.shape
    r = min(64, b)
    grid = (b // r,)
    body = functools.partial(_topk_body, k=k, r=r, d=d)
    return pl.pallas_call(
        body,
        grid=grid,
        in_specs=[pl.BlockSpec((r, d), lambda i: (i, 0))],
        out_specs=[
            pl.BlockSpec((r, d), lambda i: (i, 0)),
            pl.BlockSpec((r, k), lambda i: (i, 0)),
            pl.BlockSpec((r, k), lambda i: (i, 0)),
        ],
        out_shape=[
            jax.ShapeDtypeStruct((b, d), jnp.float32),
            jax.ShapeDtypeStruct((b, k), jnp.int32),
            jax.ShapeDtypeStruct((b, k), jnp.float32),
        ],
        scratch_shapes=[
            pltpu.VMEM((r, _QDEPTH, d // _CHUNK), jnp.float32),
            pltpu.VMEM((r, _QDEPTH, d // _CHUNK), jnp.float32),
            pltpu.VMEM((r, _QDEPTH, d // _CHUNK), jnp.int32),
        ],
        compiler_params=pltpu.CompilerParams(
            dimension_semantics=("parallel",)),
    )(dense_code)


def _decode_body(sc_ref, w_ref, out_ref):
    kk = pl.program_id(1)

    @pl.when(kk == 0)
    def _():
        out_ref[...] = jnp.zeros_like(out_ref)

    out_ref[...] = out_ref[...] + jax.lax.dot_general(
        sc_ref[...].astype(jnp.bfloat16), w_ref[...],
        (((1,), (1,)), ((), ())),
        preferred_element_type=jnp.float32)


def _decode(sparse_code, W_dec_bf16):
    b, d = sparse_code.shape
    i_dim = W_dec_bf16.shape[0]
    r = min(1024, b)
    k_tile = min(512, d)
    grid = (b // r, d // k_tile)
    return pl.pallas_call(
        _decode_body,
        grid=grid,
        in_specs=[
            pl.BlockSpec((r, k_tile), lambda i, kk: (i, kk)),
            pl.BlockSpec((i_dim, k_tile), lambda i, kk: (0, kk)),
        ],
        out_specs=pl.BlockSpec((r, i_dim), lambda i, kk: (i, 0)),
        out_shape=jax.ShapeDtypeStruct((b, i_dim), jnp.float32),
        compiler_params=pltpu.CompilerParams(
            dimension_semantics=("parallel", "arbitrary")),
    )(sparse_code, W_dec_bf16)


def kernel(x, W_enc, b_enc, W_dec):
    dense_code = _encode_matmul(x, W_enc, b_enc)
    sparse_code, active_indices, _vals = _topk(dense_code, TOPK)
    reconstructed_x = _decode(sparse_code, W_dec.astype(jnp.bfloat16))
    return reconstructed_x, sparse_code, active_indices
